# baseline (speedup 1.0000x reference)
import functools

import jax
import jax.numpy as jnp
from jax import lax
from jax.experimental import pallas as pl
from jax.experimental.pallas import tpu as pltpu
from jax.experimental.pallas import tpu_sc as plsc

N = 10000
NP = 10240
E = 320000
D = 128
NT = 12
NTP = 16

NC, NS = 2, 16
NW = NC * NS
EPW = E // NW
CH = 40
NCHUNK = EPW // CH
RPS = NP // NS

BR = 1024
GB = NP // BR


def _sc_mesh():
    return plsc.VectorSubcoreMesh(
        core_axis_name="c", subcore_axis_name="s",
        num_cores=NC, num_subcores=NS)


def _deg_body(dst_hbm, out_hbm, didx, ones_v, zb_v, deg_sh, sem):
    cid = lax.axis_index("c")
    sid = lax.axis_index("s")
    wid = cid * NS + sid

    def fill(i, _):
        zb_v[pl.ds(i * 16, 16)] = jnp.zeros((16,), jnp.float32)
        return _
    lax.fori_loop(0, RPS // 16, fill, 0)

    for off in sorted(set(list(range(0, CH - 15, 16)) + [CH - 16])):
        ones_v[pl.ds(off, 16)] = jnp.ones((16,), jnp.float32)

    pltpu.sync_copy(dst_hbm.at[wid], didx)
    pltpu.sync_copy(zb_v, deg_sh.at[pl.ds(sid * RPS, RPS)])
    plsc.subcore_barrier()

    def fire(i, _):
        pltpu.async_copy(ones_v, deg_sh.at[didx.at[i]], sem, add=True)
        return _
    lax.fori_loop(0, NCHUNK, fire, 0)

    def drain(i, _):
        pltpu.make_async_copy(ones_v, deg_sh.at[didx.at[i]], sem).wait()
        return _
    lax.fori_loop(0, NCHUNK, drain, 0)

    plsc.subcore_barrier()
    pltpu.sync_copy(deg_sh.at[pl.ds(sid * RPS, RPS)],
                    out_hbm.at[cid, pl.ds(sid * RPS, RPS)])


_deg_call = pl.kernel(
    _deg_body,
    out_type=jax.ShapeDtypeStruct((NC, NP), jnp.float32),
    mesh=_sc_mesh(),
    scratch_types=[
        pltpu.VMEM((NCHUNK, CH), jnp.int32),
        pltpu.VMEM((CH,), jnp.float32),
        pltpu.VMEM((RPS,), jnp.float32),
        pltpu.VMEM_SHARED((NP,), jnp.float32),
        pltpu.SemaphoreType.DMA,
    ],
)


NBUF = 4
NG = NCHUNK // NBUF
NREM = NCHUNK - NG * NBUF

RPA = N // NS
NZB = RPA // CH


def _agg_body(F, xp_hbm, src_hbm, dst_hbm, out_hbm, sidx, didx, rows, y_sh,
              isem, dsem, gsem, ssem):
    cid = lax.axis_index("c")
    sid = lax.axis_index("s")
    wid = cid * NS + sid
    nv = F // 16

    def sidx_start(i, b):
        pltpu.async_copy(src_hbm.at[wid, i], sidx.at[b], isem.at[b])

    def didx_start(i, b):
        pltpu.async_copy(dst_hbm.at[wid, i], didx.at[b], dsem.at[b])

    for b in range(NBUF):
        sidx_start(b, b)
        didx_start(b, b)

    def zrow(i, _):
        rows[0, i // nv, pl.ds((i % nv) * 16, 16)] = jnp.zeros((16,), jnp.float32)
        return _
    lax.fori_loop(0, CH * nv, zrow, 0)

    def zcp(i, _):
        pltpu.async_copy(rows.at[0], y_sh.at[pl.ds(sid * RPS + i * CH, CH)],
                         ssem.at[0])
        return _
    lax.fori_loop(0, RPS // CH, zcp, 0)

    def zdr(i, _):
        pltpu.make_async_copy(rows.at[0],
                              y_sh.at[pl.ds(sid * RPS + i * CH, CH)],
                              ssem.at[0]).wait()
        return _
    lax.fori_loop(0, RPS // CH, zdr, 0)
    plsc.subcore_barrier()

    def sidx_wait(i, b):
        pltpu.make_async_copy(src_hbm.at[wid, i], sidx.at[b], isem.at[b]).wait()

    def didx_wait(i, b):
        pltpu.make_async_copy(dst_hbm.at[wid, i], didx.at[b], dsem.at[b]).wait()

    def group(g, _):
        i0 = g * NBUF
        for b in range(NBUF):
            @pl.when(g > 0)
            def _drain_prev():
                pltpu.make_async_copy(rows.at[b], y_sh.at[didx.at[b]],
                                      ssem.at[b]).wait()
                didx_start(i0 + b, b)
            sidx_wait(i0 + b, b)
            pltpu.async_copy(xp_hbm.at[sidx.at[b]], rows.at[b], gsem.at[b])
        for b in range(NBUF):
            pltpu.make_async_copy(xp_hbm.at[sidx.at[b]], rows.at[b],
                                  gsem.at[b]).wait()
            didx_wait(i0 + b, b)
            pltpu.async_copy(rows.at[b], y_sh.at[didx.at[b]], ssem.at[b],
                             add=True)

            @pl.when(g + 1 < NG)
            def _next():
                sidx_start(i0 + NBUF + b, b)
        return _
    lax.fori_loop(0, NG, group, 0)

    for b in range(NBUF):
        pltpu.make_async_copy(rows.at[b], y_sh.at[didx.at[b]],
                              ssem.at[b]).wait()

    for b in range(NREM):
        i = NG * NBUF + b
        sidx_start(i, b)
        didx_start(i, b)
        sidx_wait(i, b)
        pltpu.async_copy(xp_hbm.at[sidx.at[b]], rows.at[b], gsem.at[b])
    for b in range(NREM):
        pltpu.make_async_copy(xp_hbm.at[sidx.at[b]], rows.at[b],
                              gsem.at[b]).wait()
        didx_wait(NG * NBUF + b, b)
        pltpu.async_copy(rows.at[b], y_sh.at[didx.at[b]], ssem.at[b], add=True)
    for b in range(NREM):
        pltpu.make_async_copy(rows.at[b], y_sh.at[didx.at[b]],
                              ssem.at[b]).wait()

    plsc.subcore_barrier()
    pltpu.sync_copy(y_sh.at[pl.ds(sid * RPS, RPS)],
                    out_hbm.at[cid, pl.ds(sid * RPS, RPS)])


def _make_agg(F):
    return pl.kernel(
        functools.partial(_agg_body, F),
        out_type=jax.ShapeDtypeStruct((NC, NP, F), jnp.float32),
        mesh=_sc_mesh(),
        scratch_types=[
            pltpu.VMEM((NBUF, CH), jnp.int32),
            pltpu.VMEM((NBUF, CH), jnp.int32),
            pltpu.VMEM((NBUF, CH, F), jnp.float32),
            pltpu.VMEM_SHARED((NP, F), jnp.float32),
            pltpu.SemaphoreType.DMA((NBUF,)),
            pltpu.SemaphoreType.DMA((NBUF,)),
            pltpu.SemaphoreType.DMA((NBUF,)),
            pltpu.SemaphoreType.DMA((NBUF,)),
        ],
    )


_agg128 = _make_agg(D)


def _tc1_body(deg_ref, x_ref, w1_ref, hp_ref):
    dis = lax.rsqrt(jnp.sum(deg_ref[...], axis=0) + 1.0)
    h = jnp.dot(x_ref[...], w1_ref[...], preferred_element_type=jnp.float32)
    hp_ref[...] = h * dis[:, None]


_tc1 = pl.pallas_call(
    _tc1_body,
    grid=(GB,),
    in_specs=[
        pl.BlockSpec((NC, BR), lambda i: (0, i)),
        pl.BlockSpec((BR, D), lambda i: (i, 0)),
        pl.BlockSpec((D, D), lambda i: (0, 0)),
    ],
    out_specs=pl.BlockSpec((BR, D), lambda i: (i, 0)),
    out_shape=jax.ShapeDtypeStruct((NP, D), jnp.float32),
)


def _tc2_body(deg_ref, agg_ref, hp_ref, b1_ref, rp_ref):
    dis = lax.rsqrt(jnp.sum(deg_ref[...], axis=0) + 1.0)
    s = agg_ref[0, :, :] + agg_ref[1, :, :] + hp_ref[...]
    r = jnp.maximum(s * dis[:, None] + b1_ref[...], 0.0)
    rp_ref[...] = r * dis[:, None]


_tc2 = pl.pallas_call(
    _tc2_body,
    grid=(GB,),
    in_specs=[
        pl.BlockSpec((NC, BR), lambda i: (0, i)),
        pl.BlockSpec((NC, BR, D), lambda i: (0, i, 0)),
        pl.BlockSpec((BR, D), lambda i: (i, 0)),
        pl.BlockSpec((1, D), lambda i: (0, 0)),
    ],
    out_specs=pl.BlockSpec((BR, D), lambda i: (i, 0)),
    out_shape=jax.ShapeDtypeStruct((NP, D), jnp.float32),
)


def _tc3_body(deg_ref, agg_ref, rp_ref, wt_ref, bt_ref, wc_ref, bc_ref, out_ref):
    dis = lax.rsqrt(jnp.sum(deg_ref[...], axis=0) + 1.0)
    t = agg_ref[0, :, :] + agg_ref[1, :, :] + rp_ref[...]
    z = jnp.dot(t, wt_ref[...], preferred_element_type=jnp.float32)
    s = z * dis[:, None] + bt_ref[...]
    logits = jnp.dot(s, wc_ref[...],
                     preferred_element_type=jnp.float32) + bc_ref[...]
    m = jnp.max(logits, axis=-1, keepdims=True)
    e = jnp.exp(logits - m)
    out_ref[...] = e / jnp.sum(e, axis=-1, keepdims=True)


_tc3 = pl.pallas_call(
    _tc3_body,
    grid=(GB,),
    in_specs=[
        pl.BlockSpec((NC, BR), lambda i: (0, i)),
        pl.BlockSpec((NC, BR, D), lambda i: (0, i, 0)),
        pl.BlockSpec((BR, D), lambda i: (i, 0)),
        pl.BlockSpec((D, NTP), lambda i: (0, 0)),
        pl.BlockSpec((1, NTP), lambda i: (0, 0)),
        pl.BlockSpec((NTP, NT), lambda i: (0, 0)),
        pl.BlockSpec((1, NT), lambda i: (0, 0)),
    ],
    out_specs=pl.BlockSpec((BR, NT), lambda i: (i, 0)),
    out_shape=jax.ShapeDtypeStruct((NP, NT), jnp.float32),
)


def kernel(x, edge_index, W1, b1, Wt, bt, Wc, bc):
    src = edge_index[0].reshape(NW, NCHUNK, CH)
    dst = edge_index[1].reshape(NW, NCHUNK, CH)
    x_pad = jnp.zeros((NP, D), jnp.float32).at[:N, :].set(x)
    wt_pad = jnp.pad(Wt, ((0, 0), (0, NTP - NT)))
    bt_pad = jnp.pad(bt, (0, NTP - NT))[None, :]
    wc_pad = jnp.pad(Wc, ((0, NTP - NT), (0, 0)))

    deg2 = _deg_call(dst)
    hp = _tc1(deg2, x_pad, W1)
    agg1 = _agg128(hp, src, dst)
    rp = _tc2(deg2, agg1, hp, b1[None, :])
    agg2 = _agg128(rp, src, dst)
    outp = _tc3(deg2, agg2, rp, wt_pad, bt_pad, wc_pad, bc[None, :])
    return outp[:N, :]
